# baseline (speedup 1.0000x reference)
import functools

import jax
import jax.numpy as jnp
from jax import lax
from jax.experimental import pallas as pl
from jax.experimental.pallas import tpu as pltpu
from jax.experimental.pallas import tpu_sc as plsc

N = 10000
H = 8
E = 320000
ETOT = E + N
NW = 32
B = 128
CH = 2
NB = 2 * (-(-ETOT // (2 * CH * NW * B)))
NB_TOT = CH * NB
EPAD = NW * B * NB_TOT
EPC = NW * B * NB
ROWS_PER_TILE = 8 * (-(-(N + 1) // (16 * 8)))
ACC_ROWS = 16 * ROWS_PER_TILE
R = 128
BN = 400
BE = 1024


def _proj_body(x_ref, w_ref, o_ref):
    o_ref[...] = jnp.dot(x_ref[...], w_ref[...], preferred_element_type=jnp.float32)


def _project(x, wt):
    k, r = wt.shape
    return pl.pallas_call(
        _proj_body,
        grid=(N // BN,),
        in_specs=[pl.BlockSpec((BN, k), lambda i: (i, 0)),
                  pl.BlockSpec((k, r), lambda i: (0, 0))],
        out_specs=pl.BlockSpec((BN, r), lambda i: (i, 0)),
        out_shape=jax.ShapeDtypeStruct((N, r), jnp.float32),
    )(x, wt)


def _edge1_body(xj_ref, xi_ref, m_ref, s_ref, rx_ref, o_ref):
    xj = xj_ref[...]
    xi = xi_ref[...]
    xjf = xj[:, :64]
    t = xjf * xi[:, :64]
    lg = jnp.dot(t, s_ref[...], preferred_element_type=jnp.float32)
    base = xj[:, 64:72] + xi[:, 72:80]
    sig = 1.0 / (1.0 + jnp.exp(-lg))
    a = base * sig
    a = jnp.where(a >= 0.0, a, 0.2 * a)
    ex = jnp.exp(a - m_ref[...])
    exr = jnp.dot(ex, rx_ref[...], preferred_element_type=jnp.float32)
    o_ref[...] = jnp.concatenate(
        [xjf * exr, ex, jnp.zeros((xj.shape[0], 56), jnp.float32)], axis=1)


def _edge2_body(xj_ref, xi_ref, m_ref, s_ref, al_ref, ar_ref, rx_ref,
                oa_ref, ob_ref):
    xj = xj_ref[...]
    xi = xi_ref[...]
    t = xj * xi
    lg = jnp.dot(t, s_ref[...], preferred_element_type=jnp.float32)
    base = (jnp.dot(xj, al_ref[...], preferred_element_type=jnp.float32)
            + jnp.dot(xi, ar_ref[...], preferred_element_type=jnp.float32))
    sig = 1.0 / (1.0 + jnp.exp(-lg))
    a = base * sig
    a = jnp.where(a >= 0.0, a, 0.2 * a)
    ex = jnp.exp(a - m_ref[...])
    z = jnp.zeros((xj.shape[0], 60), jnp.float32)
    exa = ex[:, :4]
    exb = ex[:, 4:]
    rxm = rx_ref[...]
    oa_ref[...] = jnp.concatenate(
        [xj[:, :64] * jnp.dot(exa, rxm, preferred_element_type=jnp.float32),
         exa, z], axis=1)
    ob_ref[...] = jnp.concatenate(
        [xj[:, 64:] * jnp.dot(exb, rxm, preferred_element_type=jnp.float32),
         exb, z], axis=1)


def _mid_body(acc_ref, b_ref, exp8_ref, w_ref, o1_ref, o2_ref):
    a = jnp.sum(acc_ref[...], axis=0)
    num = a[:, :64]
    den = a[:, 64:72]
    den_e = jnp.dot(den, exp8_ref[...], preferred_element_type=jnp.float32) + 1e-16
    hb = num / den_e + b_ref[...]
    hb = jnp.where(hb > 0, hb, jnp.exp(hb) - 1.0)
    z = jnp.dot(hb, w_ref[...], preferred_element_type=jnp.float32)
    o1_ref[...] = z[:, :128]
    o2_ref[...] = z[:, 128:]


def _post_body(aa_ref, ab_ref, b_ref, exp4_ref, mean_ref, o_ref):
    aa = jnp.sum(aa_ref[...], axis=0)
    ab = jnp.sum(ab_ref[...], axis=0)
    num_a = aa[:, :64]
    den_a = aa[:, 64:68]
    num_b = ab[:, :64]
    den_b = ab[:, 64:68]
    e4 = exp4_ref[...]
    r_a = num_a / (jnp.dot(den_a, e4, preferred_element_type=jnp.float32) + 1e-16)
    r_b = num_b / (jnp.dot(den_b, e4, preferred_element_type=jnp.float32) + 1e-16)
    mn = mean_ref[...]
    z = (jnp.dot(r_a, mn, preferred_element_type=jnp.float32)
         + jnp.dot(r_b, mn, preferred_element_type=jnp.float32) + b_ref[...])
    m = jnp.max(z, axis=1, keepdims=True)
    s = jnp.sum(jnp.exp(z - m), axis=1, keepdims=True)
    o_ref[...] = z - m - jnp.log(s)


@functools.lru_cache(maxsize=None)
def _make_gather_kernel():
    mesh = plsc.VectorSubcoreMesh(core_axis_name="c", subcore_axis_name="s")

    @functools.partial(
        pl.kernel,
        out_type=(jax.ShapeDtypeStruct((EPC, R), jnp.float32),
                  jax.ShapeDtypeStruct((EPC, R), jnp.float32)),
        mesh=mesh,
        scratch_types=[
            pltpu.VMEM((NB, B), jnp.int32),
            pltpu.VMEM((NB, B), jnp.int32),
            pltpu.VMEM((B, R), jnp.float32),
            pltpu.VMEM((B, R), jnp.float32),
            pltpu.VMEM((B, R), jnp.float32),
            pltpu.VMEM((B, R), jnp.float32),
            pltpu.SemaphoreType.DMA,
            pltpu.SemaphoreType.DMA,
            pltpu.SemaphoreType.DMA,
            pltpu.SemaphoreType.DMA,
        ],
    )
    def gather_kernel(table_hbm, src_hbm, dst_hbm, oj_hbm, oi_hbm,
                      src_v, dst_v, rj0, ri0, rj1, ri1, sg0, sg1, sw0, sw1):
        cid = lax.axis_index("c")
        sid = lax.axis_index("s")
        wid = cid * 16 + sid
        pltpu.sync_copy(src_hbm.at[wid], src_v)
        pltpu.sync_copy(dst_hbm.at[wid], dst_v)

        def g_issue(t, rj, ri, sg):
            pltpu.async_copy(table_hbm.at[src_v.at[t]], rj, sg)
            pltpu.async_copy(table_hbm.at[dst_v.at[t]], ri, sg)

        def g_drain(t, rj, ri, sg):
            pltpu.make_async_copy(table_hbm.at[src_v.at[t]], rj, sg).wait()
            pltpu.make_async_copy(table_hbm.at[dst_v.at[t]], ri, sg).wait()

        def w_issue(t, rj, ri, sw):
            base = (wid * NB + t) * B
            pltpu.async_copy(rj, oj_hbm.at[pl.ds(base, B)], sw)
            pltpu.async_copy(ri, oi_hbm.at[pl.ds(base, B)], sw)

        def w_drain(t, rj, ri, sw):
            base = (wid * NB + t) * B
            pltpu.make_async_copy(rj, oj_hbm.at[pl.ds(base, B)], sw).wait()
            pltpu.make_async_copy(ri, oi_hbm.at[pl.ds(base, B)], sw).wait()

        g_issue(0, rj0, ri0, sg0)

        def pair(jj, carry):
            t0 = 2 * jj
            t1 = t0 + 1
            g_issue(t1, rj1, ri1, sg1)
            g_drain(t0, rj0, ri0, sg0)
            w_issue(t0, rj0, ri0, sw0)
            g_drain(t1, rj1, ri1, sg1)
            w_issue(t1, rj1, ri1, sw1)
            w_drain(t0, rj0, ri0, sw0)
            g_issue(t0 + 2, rj0, ri0, sg0)
            w_drain(t1, rj1, ri1, sw1)
            return carry

        lax.fori_loop(0, NB // 2 - 1, pair, 0)
        t0 = NB - 2
        t1 = NB - 1
        g_issue(t1, rj1, ri1, sg1)
        g_drain(t0, rj0, ri0, sg0)
        w_issue(t0, rj0, ri0, sw0)
        g_drain(t1, rj1, ri1, sg1)
        w_issue(t1, rj1, ri1, sw1)
        w_drain(t0, rj0, ri0, sw0)
        w_drain(t1, rj1, ri1, sw1)

    return gather_kernel


@functools.lru_cache(maxsize=None)
def _make_scatter_kernel():
    mesh = plsc.VectorSubcoreMesh(core_axis_name="c", subcore_axis_name="s")

    @functools.partial(
        pl.kernel,
        out_type=jax.ShapeDtypeStruct((2, ACC_ROWS, R), jnp.float32),
        mesh=mesh,
        scratch_types=[
            pltpu.VMEM((NB, B), jnp.int32),
            pltpu.VMEM((B, R), jnp.float32),
            pltpu.VMEM((B, R), jnp.float32),
            pltpu.VMEM_SHARED((ACC_ROWS, R), jnp.float32),
            pltpu.SemaphoreType.DMA,
            pltpu.SemaphoreType.DMA,
            pltpu.SemaphoreType.DMA,
            pltpu.SemaphoreType.DMA,
        ],
    )
    def scatter_kernel(prod_hbm, dst_hbm, zeros_hbm, out_hbm,
                       dst_v, p0, p1, acc, sl0, sl1, sa0, sa1):
        cid = lax.axis_index("c")
        sid = lax.axis_index("s")
        wid = cid * 16 + sid
        slab = pl.ds(sid * ROWS_PER_TILE, ROWS_PER_TILE)
        pltpu.sync_copy(zeros_hbm, acc.at[slab])
        pltpu.sync_copy(dst_hbm.at[wid], dst_v)
        plsc.subcore_barrier()

        def l_issue(t, p, sl):
            base = (wid * NB + t) * B
            pltpu.async_copy(prod_hbm.at[pl.ds(base, B)], p, sl)

        def l_drain(t, p, sl):
            base = (wid * NB + t) * B
            pltpu.make_async_copy(prod_hbm.at[pl.ds(base, B)], p, sl).wait()

        def a_issue(t, p, sa):
            pltpu.async_copy(p, acc.at[dst_v.at[t]], sa, add=True)

        def a_drain(t, p, sa):
            pltpu.make_async_copy(p, acc.at[dst_v.at[t]], sa).wait()

        l_issue(0, p0, sl0)

        def pair(jj, carry):
            t0 = 2 * jj
            t1 = t0 + 1
            l_issue(t1, p1, sl1)
            l_drain(t0, p0, sl0)
            a_issue(t0, p0, sa0)
            l_drain(t1, p1, sl1)
            a_issue(t1, p1, sa1)
            a_drain(t0, p0, sa0)
            l_issue(t0 + 2, p0, sl0)
            a_drain(t1, p1, sa1)
            return carry

        lax.fori_loop(0, NB // 2 - 1, pair, 0)
        t0 = NB - 2
        t1 = NB - 1
        l_issue(t1, p1, sl1)
        l_drain(t0, p0, sl0)
        a_issue(t0, p0, sa0)
        l_drain(t1, p1, sl1)
        a_issue(t1, p1, sa1)
        a_drain(t0, p0, sa0)
        a_drain(t1, p1, sa1)
        plsc.subcore_barrier()
        pltpu.sync_copy(acc.at[slab], out_hbm.at[cid, slab])

    return scatter_kernel


def _att_block(att, C):
    a = att.reshape(H, C)
    eye = jnp.eye(H, dtype=jnp.float32)
    return (a[:, :, None] * eye[:, None, :]).reshape(H * C, H)


def _scatter(prod, dst_p):
    zeros = jnp.zeros((ROWS_PER_TILE, R), jnp.float32)
    return _make_scatter_kernel()(prod, dst_p, zeros)


def kernel(x, edge_index, W1, att_l1, att_r1, b1, W2, att_l2, att_r2, b2):
    src, dst = edge_index[0], edge_index[1]
    loop = jnp.arange(N, dtype=src.dtype)
    valid = jnp.concatenate([src != dst, jnp.ones((N,), bool)])
    src_all = jnp.concatenate([src, loop])
    dst_all = jnp.concatenate([dst, loop])
    dst_g = jnp.where(valid, dst_all, 0)
    dst_s = jnp.where(valid, dst_all, N)
    pad = EPAD - ETOT
    src_p = jnp.concatenate([src_all, jnp.zeros((pad,), src.dtype)]).reshape(NW, NB_TOT, B)
    dstg_p = jnp.concatenate([dst_g, jnp.zeros((pad,), src.dtype)]).reshape(NW, NB_TOT, B)
    dsts_p = jnp.concatenate([dst_s, jnp.full((pad,), N, src.dtype)]).reshape(NW, NB_TOT, B)
    chunk = lambda a, c: a[:, c * NB:(c + 1) * NB, :]

    W1T = W1.T
    wt1 = jnp.concatenate(
        [W1T, W1T @ _att_block(att_l1, 8), W1T @ _att_block(att_r1, 8),
         jnp.zeros((128, 48), jnp.float32)], axis=1)
    W2T = W2.T
    wt2 = jnp.concatenate(
        [W2T, W2T @ _att_block(att_l2, 16), W2T @ _att_block(att_r2, 16)],
        axis=1)
    al2 = _att_block(att_l2, 16)
    ar2 = _att_block(att_r2, 16)

    table1 = _project(x, wt1)
    m1 = jnp.maximum(jnp.max(table1[:, 64:72], 0) + jnp.max(table1[:, 72:80], 0), 0.0)
    seg1 = jnp.kron(jnp.eye(8), jnp.ones((8, 1))).astype(jnp.float32)
    rx1 = jnp.kron(jnp.eye(8), jnp.ones((1, 8))).astype(jnp.float32)
    acc1_parts = []
    for c in range(CH):
        rows_j1, rows_i1 = _make_gather_kernel()(
            table1, chunk(src_p, c), chunk(dstg_p, c))
        prod1 = pl.pallas_call(
            _edge1_body,
            grid=(EPC // BE,),
            in_specs=[pl.BlockSpec((BE, R), lambda i: (i, 0)),
                      pl.BlockSpec((BE, R), lambda i: (i, 0)),
                      pl.BlockSpec((1, 8), lambda i: (0, 0)),
                      pl.BlockSpec((64, 8), lambda i: (0, 0)),
                      pl.BlockSpec((8, 64), lambda i: (0, 0))],
            out_specs=pl.BlockSpec((BE, R), lambda i: (i, 0)),
            out_shape=jax.ShapeDtypeStruct((EPC, R), jnp.float32),
        )(rows_j1, rows_i1, m1.reshape(1, 8), seg1, rx1)
        acc1_parts.append(_scatter(prod1, chunk(dsts_p, c)))
    acc1 = jnp.concatenate(acc1_parts, axis=0)

    exp8 = jnp.kron(jnp.eye(8), jnp.ones((1, 8))).astype(jnp.float32)
    table2, att2 = pl.pallas_call(
        _mid_body,
        grid=(N // BN,),
        in_specs=[pl.BlockSpec((2 * CH, BN, R), lambda i: (0, i, 0)),
                  pl.BlockSpec((1, 64), lambda i: (0, 0)),
                  pl.BlockSpec((8, 64), lambda i: (0, 0)),
                  pl.BlockSpec((64, 144), lambda i: (0, 0))],
        out_specs=[pl.BlockSpec((BN, 128), lambda i: (i, 0)),
                   pl.BlockSpec((BN, 16), lambda i: (i, 0))],
        out_shape=[jax.ShapeDtypeStruct((N, 128), jnp.float32),
                   jax.ShapeDtypeStruct((N, 16), jnp.float32)],
    )(acc1, b1.reshape(1, 64), exp8, wt2)

    m2 = jnp.maximum(jnp.max(att2[:, :8], 0) + jnp.max(att2[:, 8:], 0), 0.0)
    seg2 = jnp.kron(jnp.eye(8), jnp.ones((16, 1))).astype(jnp.float32)
    rx2 = jnp.kron(jnp.eye(4), jnp.ones((1, 16))).astype(jnp.float32)
    acc2a_parts = []
    acc2b_parts = []
    for c in range(CH):
        rows_j2, rows_i2 = _make_gather_kernel()(
            table2, chunk(src_p, c), chunk(dstg_p, c))
        prod2a, prod2b = pl.pallas_call(
            _edge2_body,
            grid=(EPC // BE,),
            in_specs=[pl.BlockSpec((BE, R), lambda i: (i, 0)),
                      pl.BlockSpec((BE, R), lambda i: (i, 0)),
                      pl.BlockSpec((1, 8), lambda i: (0, 0)),
                      pl.BlockSpec((128, 8), lambda i: (0, 0)),
                      pl.BlockSpec((128, 8), lambda i: (0, 0)),
                      pl.BlockSpec((128, 8), lambda i: (0, 0)),
                      pl.BlockSpec((4, 64), lambda i: (0, 0))],
            out_specs=[pl.BlockSpec((BE, R), lambda i: (i, 0)),
                       pl.BlockSpec((BE, R), lambda i: (i, 0))],
            out_shape=[jax.ShapeDtypeStruct((EPC, R), jnp.float32),
                       jax.ShapeDtypeStruct((EPC, R), jnp.float32)],
        )(rows_j2, rows_i2, m2.reshape(1, 8), seg2, al2, ar2, rx2)
        acc2a_parts.append(_scatter(prod2a, chunk(dsts_p, c)))
        acc2b_parts.append(_scatter(prod2b, chunk(dsts_p, c)))
    acc2a = jnp.concatenate(acc2a_parts, axis=0)
    acc2b = jnp.concatenate(acc2b_parts, axis=0)

    exp4 = jnp.kron(jnp.eye(4), jnp.ones((1, 16))).astype(jnp.float32)
    meanm = (jnp.kron(jnp.ones((4, 1)), jnp.eye(16)) / 8.0).astype(jnp.float32)
    logp = pl.pallas_call(
        _post_body,
        grid=(N // BN,),
        in_specs=[pl.BlockSpec((2 * CH, BN, R), lambda i: (0, i, 0)),
                  pl.BlockSpec((2 * CH, BN, R), lambda i: (0, i, 0)),
                  pl.BlockSpec((1, 16), lambda i: (0, 0)),
                  pl.BlockSpec((4, 64), lambda i: (0, 0)),
                  pl.BlockSpec((64, 16), lambda i: (0, 0))],
        out_specs=pl.BlockSpec((BN, 16), lambda i: (i, 0)),
        out_shape=jax.ShapeDtypeStruct((N, 16), jnp.float32),
    )(acc2a, acc2b, b2.reshape(1, 16), exp4, meanm)

    return logp, jnp.zeros((), jnp.float32)


# speedup vs baseline: 1.6808x; 1.6808x over previous
import functools

import jax
import jax.numpy as jnp
from jax import lax
from jax.experimental import pallas as pl
from jax.experimental.pallas import tpu as pltpu
from jax.experimental.pallas import tpu_sc as plsc

N = 10000
H = 8
E = 320000
ETOT = E + N
NW = 32
B = 128
CH = 1
NB = 2 * (-(-ETOT // (2 * CH * NW * B)))
NB_TOT = CH * NB
EPAD = NW * B * NB_TOT
EPC = NW * B * NB
ROWS_PER_TILE = 8 * (-(-(N + 1) // (16 * 8)))
ACC_ROWS = 16 * ROWS_PER_TILE
R = 128
BN = 400
BE = 1024


def _proj_body(x_ref, w_ref, o_ref):
    o_ref[...] = jnp.dot(x_ref[...], w_ref[...], preferred_element_type=jnp.float32)


def _project(x, wt):
    k, r = wt.shape
    return pl.pallas_call(
        _proj_body,
        grid=(N // BN,),
        in_specs=[pl.BlockSpec((BN, k), lambda i: (i, 0)),
                  pl.BlockSpec((k, r), lambda i: (0, 0))],
        out_specs=pl.BlockSpec((BN, r), lambda i: (i, 0)),
        out_shape=jax.ShapeDtypeStruct((N, r), jnp.float32),
    )(x, wt)


def _edge1_body(xj_ref, xi_ref, m_ref, s_ref, rx_ref, o_ref):
    xj = xj_ref[...]
    xi = xi_ref[...]
    xjf = xj[:, :64]
    t = xjf * xi[:, :64]
    lg = jnp.dot(t, s_ref[...], preferred_element_type=jnp.float32)
    base = xj[:, 64:72] + xi[:, 72:80]
    sig = 1.0 / (1.0 + jnp.exp(-lg))
    a = base * sig
    a = jnp.where(a >= 0.0, a, 0.2 * a)
    ex = jnp.exp(a - m_ref[...])
    exr = jnp.dot(ex, rx_ref[...], preferred_element_type=jnp.float32)
    o_ref[...] = jnp.concatenate(
        [xjf * exr, ex, jnp.zeros((xj.shape[0], 56), jnp.float32)], axis=1)


def _edge2_body(xj_ref, xi_ref, m_ref, s_ref, al_ref, ar_ref, rx_ref,
                oa_ref, ob_ref):
    xj = xj_ref[...]
    xi = xi_ref[...]
    t = xj * xi
    lg = jnp.dot(t, s_ref[...], preferred_element_type=jnp.float32)
    base = (jnp.dot(xj, al_ref[...], preferred_element_type=jnp.float32)
            + jnp.dot(xi, ar_ref[...], preferred_element_type=jnp.float32))
    sig = 1.0 / (1.0 + jnp.exp(-lg))
    a = base * sig
    a = jnp.where(a >= 0.0, a, 0.2 * a)
    ex = jnp.exp(a - m_ref[...])
    z = jnp.zeros((xj.shape[0], 60), jnp.float32)
    exa = ex[:, :4]
    exb = ex[:, 4:]
    rxm = rx_ref[...]
    oa_ref[...] = jnp.concatenate(
        [xj[:, :64] * jnp.dot(exa, rxm, preferred_element_type=jnp.float32),
         exa, z], axis=1)
    ob_ref[...] = jnp.concatenate(
        [xj[:, 64:] * jnp.dot(exb, rxm, preferred_element_type=jnp.float32),
         exb, z], axis=1)


def _mid_body(acc_ref, b_ref, exp8_ref, w_ref, o1_ref, o2_ref):
    a = jnp.sum(acc_ref[...], axis=0)
    num = a[:, :64]
    den = a[:, 64:72]
    den_e = jnp.dot(den, exp8_ref[...], preferred_element_type=jnp.float32) + 1e-16
    hb = num / den_e + b_ref[...]
    hb = jnp.where(hb > 0, hb, jnp.exp(hb) - 1.0)
    z = jnp.dot(hb, w_ref[...], preferred_element_type=jnp.float32)
    o1_ref[...] = z[:, :128]
    o2_ref[...] = z[:, 128:]


def _post_body(aa_ref, ab_ref, b_ref, exp4_ref, mean_ref, o_ref):
    aa = jnp.sum(aa_ref[...], axis=0)
    ab = jnp.sum(ab_ref[...], axis=0)
    num_a = aa[:, :64]
    den_a = aa[:, 64:68]
    num_b = ab[:, :64]
    den_b = ab[:, 64:68]
    e4 = exp4_ref[...]
    r_a = num_a / (jnp.dot(den_a, e4, preferred_element_type=jnp.float32) + 1e-16)
    r_b = num_b / (jnp.dot(den_b, e4, preferred_element_type=jnp.float32) + 1e-16)
    mn = mean_ref[...]
    z = (jnp.dot(r_a, mn, preferred_element_type=jnp.float32)
         + jnp.dot(r_b, mn, preferred_element_type=jnp.float32) + b_ref[...])
    m = jnp.max(z, axis=1, keepdims=True)
    s = jnp.sum(jnp.exp(z - m), axis=1, keepdims=True)
    o_ref[...] = z - m - jnp.log(s)


@functools.lru_cache(maxsize=None)
def _make_gather_kernel():
    mesh = plsc.VectorSubcoreMesh(core_axis_name="c", subcore_axis_name="s")

    @functools.partial(
        pl.kernel,
        out_type=(jax.ShapeDtypeStruct((EPC, R), jnp.float32),
                  jax.ShapeDtypeStruct((EPC, R), jnp.float32)),
        mesh=mesh,
        scratch_types=[
            pltpu.VMEM((NB, B), jnp.int32),
            pltpu.VMEM((NB, B), jnp.int32),
            pltpu.VMEM((B, R), jnp.float32),
            pltpu.VMEM((B, R), jnp.float32),
            pltpu.VMEM((B, R), jnp.float32),
            pltpu.VMEM((B, R), jnp.float32),
            pltpu.SemaphoreType.DMA,
            pltpu.SemaphoreType.DMA,
            pltpu.SemaphoreType.DMA,
            pltpu.SemaphoreType.DMA,
        ],
    )
    def gather_kernel(table_hbm, src_hbm, dst_hbm, oj_hbm, oi_hbm,
                      src_v, dst_v, rj0, ri0, rj1, ri1, sg0, sg1, sw0, sw1):
        cid = lax.axis_index("c")
        sid = lax.axis_index("s")
        wid = cid * 16 + sid
        pltpu.sync_copy(src_hbm.at[wid], src_v)
        pltpu.sync_copy(dst_hbm.at[wid], dst_v)

        def g_issue(t, rj, ri, sg):
            pltpu.async_copy(table_hbm.at[src_v.at[t]], rj, sg)
            pltpu.async_copy(table_hbm.at[dst_v.at[t]], ri, sg)

        def g_drain(t, rj, ri, sg):
            pltpu.make_async_copy(table_hbm.at[src_v.at[t]], rj, sg).wait()
            pltpu.make_async_copy(table_hbm.at[dst_v.at[t]], ri, sg).wait()

        def w_issue(t, rj, ri, sw):
            base = (wid * NB + t) * B
            pltpu.async_copy(rj, oj_hbm.at[pl.ds(base, B)], sw)
            pltpu.async_copy(ri, oi_hbm.at[pl.ds(base, B)], sw)

        def w_drain(t, rj, ri, sw):
            base = (wid * NB + t) * B
            pltpu.make_async_copy(rj, oj_hbm.at[pl.ds(base, B)], sw).wait()
            pltpu.make_async_copy(ri, oi_hbm.at[pl.ds(base, B)], sw).wait()

        g_issue(0, rj0, ri0, sg0)

        def pair(jj, carry):
            t0 = 2 * jj
            t1 = t0 + 1
            g_issue(t1, rj1, ri1, sg1)
            g_drain(t0, rj0, ri0, sg0)
            w_issue(t0, rj0, ri0, sw0)
            g_drain(t1, rj1, ri1, sg1)
            w_issue(t1, rj1, ri1, sw1)
            w_drain(t0, rj0, ri0, sw0)
            g_issue(t0 + 2, rj0, ri0, sg0)
            w_drain(t1, rj1, ri1, sw1)
            return carry

        lax.fori_loop(0, NB // 2 - 1, pair, 0)
        t0 = NB - 2
        t1 = NB - 1
        g_issue(t1, rj1, ri1, sg1)
        g_drain(t0, rj0, ri0, sg0)
        w_issue(t0, rj0, ri0, sw0)
        g_drain(t1, rj1, ri1, sg1)
        w_issue(t1, rj1, ri1, sw1)
        w_drain(t0, rj0, ri0, sw0)
        w_drain(t1, rj1, ri1, sw1)

    return gather_kernel


@functools.lru_cache(maxsize=None)
def _make_scatter_kernel():
    mesh = plsc.VectorSubcoreMesh(core_axis_name="c", subcore_axis_name="s")

    @functools.partial(
        pl.kernel,
        out_type=jax.ShapeDtypeStruct((2, ACC_ROWS, R), jnp.float32),
        mesh=mesh,
        scratch_types=[
            pltpu.VMEM((NB, B), jnp.int32),
            pltpu.VMEM((B, R), jnp.float32),
            pltpu.VMEM((B, R), jnp.float32),
            pltpu.VMEM_SHARED((ACC_ROWS, R), jnp.float32),
            pltpu.SemaphoreType.DMA,
            pltpu.SemaphoreType.DMA,
            pltpu.SemaphoreType.DMA,
            pltpu.SemaphoreType.DMA,
        ],
    )
    def scatter_kernel(prod_hbm, dst_hbm, zeros_hbm, out_hbm,
                       dst_v, p0, p1, acc, sl0, sl1, sa0, sa1):
        cid = lax.axis_index("c")
        sid = lax.axis_index("s")
        wid = cid * 16 + sid
        slab = pl.ds(sid * ROWS_PER_TILE, ROWS_PER_TILE)
        pltpu.sync_copy(zeros_hbm, acc.at[slab])
        pltpu.sync_copy(dst_hbm.at[wid], dst_v)
        plsc.subcore_barrier()

        def l_issue(t, p, sl):
            base = (wid * NB + t) * B
            pltpu.async_copy(prod_hbm.at[pl.ds(base, B)], p, sl)

        def l_drain(t, p, sl):
            base = (wid * NB + t) * B
            pltpu.make_async_copy(prod_hbm.at[pl.ds(base, B)], p, sl).wait()

        def a_issue(t, p, sa):
            pltpu.async_copy(p, acc.at[dst_v.at[t]], sa, add=True)

        def a_drain(t, p, sa):
            pltpu.make_async_copy(p, acc.at[dst_v.at[t]], sa).wait()

        l_issue(0, p0, sl0)

        def pair(jj, carry):
            t0 = 2 * jj
            t1 = t0 + 1
            l_issue(t1, p1, sl1)
            l_drain(t0, p0, sl0)
            a_issue(t0, p0, sa0)
            l_drain(t1, p1, sl1)
            a_issue(t1, p1, sa1)
            a_drain(t0, p0, sa0)
            l_issue(t0 + 2, p0, sl0)
            a_drain(t1, p1, sa1)
            return carry

        lax.fori_loop(0, NB // 2 - 1, pair, 0)
        t0 = NB - 2
        t1 = NB - 1
        l_issue(t1, p1, sl1)
        l_drain(t0, p0, sl0)
        a_issue(t0, p0, sa0)
        l_drain(t1, p1, sl1)
        a_issue(t1, p1, sa1)
        a_drain(t0, p0, sa0)
        a_drain(t1, p1, sa1)
        plsc.subcore_barrier()
        pltpu.sync_copy(acc.at[slab], out_hbm.at[cid, slab])

    return scatter_kernel


def _att_block(att, C):
    a = att.reshape(H, C)
    eye = jnp.eye(H, dtype=jnp.float32)
    return (a[:, :, None] * eye[:, None, :]).reshape(H * C, H)


def _scatter(prod, dst_p):
    zeros = jnp.zeros((ROWS_PER_TILE, R), jnp.float32)
    return _make_scatter_kernel()(prod, dst_p, zeros)


def kernel(x, edge_index, W1, att_l1, att_r1, b1, W2, att_l2, att_r2, b2):
    src, dst = edge_index[0], edge_index[1]
    loop = jnp.arange(N, dtype=src.dtype)
    valid = jnp.concatenate([src != dst, jnp.ones((N,), bool)])
    src_all = jnp.concatenate([src, loop])
    dst_all = jnp.concatenate([dst, loop])
    dst_g = jnp.where(valid, dst_all, 0)
    dst_s = jnp.where(valid, dst_all, N)
    pad = EPAD - ETOT
    src_p = jnp.concatenate([src_all, jnp.zeros((pad,), src.dtype)]).reshape(NW, NB_TOT, B)
    dstg_p = jnp.concatenate([dst_g, jnp.zeros((pad,), src.dtype)]).reshape(NW, NB_TOT, B)
    dsts_p = jnp.concatenate([dst_s, jnp.full((pad,), N, src.dtype)]).reshape(NW, NB_TOT, B)
    chunk = lambda a, c: a[:, c * NB:(c + 1) * NB, :]

    W1T = W1.T
    wt1 = jnp.concatenate(
        [W1T, W1T @ _att_block(att_l1, 8), W1T @ _att_block(att_r1, 8),
         jnp.zeros((128, 48), jnp.float32)], axis=1)
    W2T = W2.T
    wt2 = jnp.concatenate(
        [W2T, W2T @ _att_block(att_l2, 16), W2T @ _att_block(att_r2, 16)],
        axis=1)
    al2 = _att_block(att_l2, 16)
    ar2 = _att_block(att_r2, 16)

    table1 = _project(x, wt1)
    m1 = jnp.maximum(jnp.max(table1[:, 64:72], 0) + jnp.max(table1[:, 72:80], 0), 0.0)
    seg1 = jnp.kron(jnp.eye(8), jnp.ones((8, 1))).astype(jnp.float32)
    rx1 = jnp.kron(jnp.eye(8), jnp.ones((1, 8))).astype(jnp.float32)
    acc1_parts = []
    for c in range(CH):
        rows_j1, rows_i1 = _make_gather_kernel()(
            table1, chunk(src_p, c), chunk(dstg_p, c))
        prod1 = pl.pallas_call(
            _edge1_body,
            grid=(EPC // BE,),
            in_specs=[pl.BlockSpec((BE, R), lambda i: (i, 0)),
                      pl.BlockSpec((BE, R), lambda i: (i, 0)),
                      pl.BlockSpec((1, 8), lambda i: (0, 0)),
                      pl.BlockSpec((64, 8), lambda i: (0, 0)),
                      pl.BlockSpec((8, 64), lambda i: (0, 0))],
            out_specs=pl.BlockSpec((BE, R), lambda i: (i, 0)),
            out_shape=jax.ShapeDtypeStruct((EPC, R), jnp.float32),
        )(rows_j1, rows_i1, m1.reshape(1, 8), seg1, rx1)
        acc1_parts.append(_scatter(prod1, chunk(dsts_p, c)))
    acc1 = jnp.concatenate(acc1_parts, axis=0)

    exp8 = jnp.kron(jnp.eye(8), jnp.ones((1, 8))).astype(jnp.float32)
    table2, att2 = pl.pallas_call(
        _mid_body,
        grid=(N // BN,),
        in_specs=[pl.BlockSpec((2 * CH, BN, R), lambda i: (0, i, 0)),
                  pl.BlockSpec((1, 64), lambda i: (0, 0)),
                  pl.BlockSpec((8, 64), lambda i: (0, 0)),
                  pl.BlockSpec((64, 144), lambda i: (0, 0))],
        out_specs=[pl.BlockSpec((BN, 128), lambda i: (i, 0)),
                   pl.BlockSpec((BN, 16), lambda i: (i, 0))],
        out_shape=[jax.ShapeDtypeStruct((N, 128), jnp.float32),
                   jax.ShapeDtypeStruct((N, 16), jnp.float32)],
    )(acc1, b1.reshape(1, 64), exp8, wt2)

    m2 = jnp.maximum(jnp.max(att2[:, :8], 0) + jnp.max(att2[:, 8:], 0), 0.0)
    seg2 = jnp.kron(jnp.eye(8), jnp.ones((16, 1))).astype(jnp.float32)
    rx2 = jnp.kron(jnp.eye(4), jnp.ones((1, 16))).astype(jnp.float32)
    acc2a_parts = []
    acc2b_parts = []
    for c in range(CH):
        rows_j2, rows_i2 = _make_gather_kernel()(
            table2, chunk(src_p, c), chunk(dstg_p, c))
        prod2a, prod2b = pl.pallas_call(
            _edge2_body,
            grid=(EPC // BE,),
            in_specs=[pl.BlockSpec((BE, R), lambda i: (i, 0)),
                      pl.BlockSpec((BE, R), lambda i: (i, 0)),
                      pl.BlockSpec((1, 8), lambda i: (0, 0)),
                      pl.BlockSpec((128, 8), lambda i: (0, 0)),
                      pl.BlockSpec((128, 8), lambda i: (0, 0)),
                      pl.BlockSpec((128, 8), lambda i: (0, 0)),
                      pl.BlockSpec((4, 64), lambda i: (0, 0))],
            out_specs=[pl.BlockSpec((BE, R), lambda i: (i, 0)),
                       pl.BlockSpec((BE, R), lambda i: (i, 0))],
            out_shape=[jax.ShapeDtypeStruct((EPC, R), jnp.float32),
                       jax.ShapeDtypeStruct((EPC, R), jnp.float32)],
        )(rows_j2, rows_i2, m2.reshape(1, 8), seg2, al2, ar2, rx2)
        acc2a_parts.append(_scatter(prod2a, chunk(dsts_p, c)))
        acc2b_parts.append(_scatter(prod2b, chunk(dsts_p, c)))
    acc2a = jnp.concatenate(acc2a_parts, axis=0)
    acc2b = jnp.concatenate(acc2b_parts, axis=0)

    exp4 = jnp.kron(jnp.eye(4), jnp.ones((1, 16))).astype(jnp.float32)
    meanm = (jnp.kron(jnp.ones((4, 1)), jnp.eye(16)) / 8.0).astype(jnp.float32)
    logp = pl.pallas_call(
        _post_body,
        grid=(N // BN,),
        in_specs=[pl.BlockSpec((2 * CH, BN, R), lambda i: (0, i, 0)),
                  pl.BlockSpec((2 * CH, BN, R), lambda i: (0, i, 0)),
                  pl.BlockSpec((1, 16), lambda i: (0, 0)),
                  pl.BlockSpec((4, 64), lambda i: (0, 0)),
                  pl.BlockSpec((64, 16), lambda i: (0, 0))],
        out_specs=pl.BlockSpec((BN, 16), lambda i: (i, 0)),
        out_shape=jax.ShapeDtypeStruct((N, 16), jnp.float32),
    )(acc2a, acc2b, b2.reshape(1, 16), exp4, meanm)

    return logp, jnp.zeros((), jnp.float32)
